# baseline (speedup 1.0000x reference)
import functools

import jax
import jax.numpy as jnp
from jax import lax
from jax.experimental import pallas as pl
from jax.experimental.pallas import tpu as pltpu
from jax.experimental.pallas import tpu_sc as plsc

_N, _S, _C, _K, _D = 8, 576, 1, 1024, 256
_NS = _N * _S
_R = 512
_G = _NS // _R
_NCAND = 3
_B = _NCAND * _NS
_NW = 32
_BPW = _B // _NW
_CHUNK_I = 6
_IPC = _BPW // _CHUNK_I


def _screen_kernel(x_ref, e_ref, i1_ref, i2_ref, i3_ref):
    x = x_ref[...]
    e1 = e_ref[...].astype(jnp.bfloat16)
    xe = jax.lax.dot_general(
        x.astype(jnp.bfloat16), e1,
        (((1,), (1,)), ((), ())), preferred_element_type=jnp.float32)
    s = -2.0 * xe
    iota = jax.lax.broadcasted_iota(jnp.int32, (_R, _K), 1)
    big = jnp.float32(3.0e38)
    outs = [i1_ref, i2_ref, i3_ref]
    for r in outs:
        m = jnp.min(s, axis=1)[:, None]
        ij = jnp.min(jnp.where(s == m, iota, _K), axis=1)[:, None]
        r[...] = ij
        s = jnp.where(iota == ij, big, s)


def _screen(x2, e2):
    out_shapes = tuple(
        jax.ShapeDtypeStruct((_NS, 1), jnp.int32) for _ in range(_NCAND))
    return pl.pallas_call(
        _screen_kernel,
        grid=(_G,),
        in_specs=[
            pl.BlockSpec((_R, _D), lambda i: (i, 0)),
            pl.BlockSpec((_K, _D), lambda i: (0, 0)),
        ],
        out_specs=tuple(
            pl.BlockSpec((_R, 1), lambda i: (i, 0)) for _ in range(_NCAND)),
        out_shape=out_shapes,
    )(x2, e2)


def _sc_gather(table, idx):
    mesh = plsc.VectorSubcoreMesh(core_axis_name="c", subcore_axis_name="s")

    @functools.partial(
        pl.kernel,
        out_type=jax.ShapeDtypeStruct((_B, _D), jnp.float32),
        mesh=mesh,
        scratch_types=[
            pltpu.VMEM((_CHUNK_I, _IPC), jnp.int32),
            pltpu.VMEM((_BPW, _D), jnp.float32),
            pltpu.SemaphoreType.DMA,
        ],
    )
    def k(table_hbm, idx_hbm, out_hbm, idx_v, rows_v, sem):
        wid = lax.axis_index("s") * 2 + lax.axis_index("c")
        base = wid * _BPW
        pltpu.sync_copy(idx_hbm.at[wid], idx_v)
        copies = []
        for j in range(_CHUNK_I):
            copies.append(pltpu.async_copy(
                table_hbm.at[idx_v.at[j]],
                rows_v.at[pl.ds(j * _IPC, _IPC)], sem))
        for c in copies:
            c.wait()
        pltpu.sync_copy(rows_v, out_hbm.at[pl.ds(base, _BPW)])

    return k(table, idx.reshape(_NW, _CHUNK_I, _IPC))


def _finish_kernel(x_ref, g1_ref, g2_ref, g3_ref, i1_ref, i2_ref, i3_ref,
                   out0_ref, out1_ref, hist_ref, ent_ref):
    i = pl.program_id(0)
    x = x_ref[...]

    best_d = best_i = best_e = None
    for g_ref, i_ref in ((g1_ref, i1_ref), (g2_ref, i2_ref), (g3_ref, i3_ref)):
        ev = g_ref[...]
        ij = i_ref[...]
        diff = x - ev
        sq = diff * diff
        dp = sq[:, :128] + sq[:, 128:]
        acc = dp
        for j in range(1, 16):
            acc = acc + jnp.roll(dp, -8 * j, axis=1)
        b = acc + jnp.roll(acc, -4, axis=1)
        c = b + jnp.roll(b, -2, axis=1)
        t = c + jnp.roll(c, -1, axis=1)
        d2 = t[:, 0:1]
        d = d2 * jax.lax.rsqrt(d2)
        if best_d is None:
            best_d, best_i, best_e = d, ij, ev
        else:
            take = (d < best_d) | ((d == best_d) & (ij < best_i))
            best_d = jnp.where(take, d, best_d)
            best_i = jnp.where(take, ij, best_i)
            best_e = jnp.where(take, ev, best_e)

    out0_ref[...] = (best_e - x) + x
    out1_ref[...] = best_d * best_d

    iota = jax.lax.broadcasted_iota(jnp.int32, (_R, _K), 1)
    ohw = (iota == best_i).astype(jnp.float32)
    h = jnp.sum(ohw, axis=0, keepdims=True)

    @pl.when(i == 0)
    def _():
        hist_ref[...] = h

    @pl.when(i != 0)
    def _():
        hist_ref[...] = hist_ref[...] + h

    @pl.when(i == _G - 1)
    def _():
        hist = hist_ref[...]
        prob = hist / jnp.float32(_NS)
        ent = -jnp.sum(jnp.where(hist > 0,
                                 prob * jnp.log(jnp.where(hist > 0, prob, 1.0)),
                                 0.0))
        ent_ref[...] = ent.reshape(1, 1)


def _finish(x2, g1, g2, g3, i1, i2, i3):
    out_shapes = (
        jax.ShapeDtypeStruct((_NS, _D), jnp.float32),
        jax.ShapeDtypeStruct((_NS, 1), jnp.float32),
        jax.ShapeDtypeStruct((1, _K), jnp.float32),
        jax.ShapeDtypeStruct((1, 1), jnp.float32),
    )
    row_spec = pl.BlockSpec((_R, _D), lambda i: (i, 0))
    idx_spec = pl.BlockSpec((_R, 1), lambda i: (i, 0))
    return pl.pallas_call(
        _finish_kernel,
        grid=(_G,),
        in_specs=[row_spec, row_spec, row_spec, row_spec,
                  idx_spec, idx_spec, idx_spec],
        out_specs=(
            row_spec,
            idx_spec,
            pl.BlockSpec((1, _K), lambda i: (0, 0)),
            pl.BlockSpec((1, 1), lambda i: (0, 0)),
        ),
        out_shape=out_shapes,
        compiler_params=pltpu.CompilerParams(
            dimension_semantics=("arbitrary",)),
    )(x2, g1, g2, g3, i1, i2, i3)


def kernel(x0, embedding0):
    x2 = x0.reshape(_NS, _D)
    e2 = embedding0.reshape(_K, _D)
    i1, i2, i3 = _screen(x2, e2)
    idx = jnp.concatenate([i1, i2, i3], axis=0).reshape(_B)
    g = _sc_gather(e2, idx)
    g1, g2, g3 = g[:_NS], g[_NS:2 * _NS], g[2 * _NS:]
    o0, o1, _hist, ent = _finish(x2, g1, g2, g3, i1, i2, i3)
    out0 = o0.reshape(_N, _S, _C, _D)
    out1 = o1.reshape(_N, _S, _C)
    entropy = ent[0, 0]
    return (out0, out1, out1, entropy)


# speedup vs baseline: 1.3519x; 1.3519x over previous
import jax
import jax.numpy as jnp
from jax.experimental import pallas as pl
from jax.experimental.pallas import tpu as pltpu

_N, _S, _C, _K, _D = 8, 576, 1, 1024, 256
_NS = _N * _S
_R = 512
_G = _NS // _R


def _vq_kernel(x_ref, e1_ref, e2_ref, e3_ref, out0_ref, out1_ref, hist_ref,
               ent_ref):
    i = pl.program_id(0)
    x = x_ref[...]
    e1 = e1_ref[...]
    e2 = e2_ref[...]
    e3 = e3_ref[...]

    xe = jax.lax.dot_general(
        x.astype(jnp.bfloat16), e1,
        (((1,), (1,)), ((), ())), preferred_element_type=jnp.float32)
    s = -2.0 * xe

    iota = jax.lax.broadcasted_iota(jnp.int32, (_R, _K), 1)
    big = jnp.float32(3.0e38)

    cand = []
    for _ in range(3):
        m = jnp.min(s, axis=1)[:, None]
        ij = jnp.min(jnp.where(s == m, iota, _K), axis=1)[:, None]
        cand.append(ij)
        s = jnp.where(iota == ij, big, s)

    best_d = best_i = best_e = None
    ohs = []
    for ij in cand:
        oh = (iota == ij).astype(jnp.bfloat16)
        ohs.append(oh)
        dn = (((1,), (0,)), ((), ()))
        ev = (jax.lax.dot_general(oh, e1, dn, preferred_element_type=jnp.float32)
              + jax.lax.dot_general(oh, e2, dn, preferred_element_type=jnp.float32)
              ) + jax.lax.dot_general(oh, e3, dn, preferred_element_type=jnp.float32)
        diff = x - ev
        sq = diff * diff
        dp = sq[:, :128] + sq[:, 128:]
        acc = dp
        for j in range(1, 16):
            acc = acc + jnp.roll(dp, -8 * j, axis=1)
        b = acc + jnp.roll(acc, -4, axis=1)
        c = b + jnp.roll(b, -2, axis=1)
        t = c + jnp.roll(c, -1, axis=1)
        d2 = t[:, 0:1]
        d = d2 * jax.lax.rsqrt(d2)
        if best_d is None:
            best_d, best_i, best_e = d, ij, ev
        else:
            take = (d < best_d) | ((d == best_d) & (ij < best_i))
            best_d = jnp.where(take, d, best_d)
            best_i = jnp.where(take, ij, best_i)
            best_e = jnp.where(take, ev, best_e)

    out0_ref[...] = (best_e - x) + x
    out1_ref[...] = best_d * best_d

    hn = (((0,), (0,)), ((), ()))
    h = jnp.zeros((1, _K), jnp.float32)
    for ij, oh in zip(cand, ohs):
        w = (ij == best_i).astype(jnp.bfloat16)
        h = h + jax.lax.dot_general(
            w, oh, hn, preferred_element_type=jnp.float32)

    @pl.when(i == 0)
    def _():
        hist_ref[...] = h

    @pl.when(i != 0)
    def _():
        hist_ref[...] = hist_ref[...] + h

    @pl.when(i == _G - 1)
    def _():
        hist = hist_ref[...]
        prob = hist / jnp.float32(_NS)
        ent = -jnp.sum(jnp.where(hist > 0,
                                 prob * jnp.log(jnp.where(hist > 0, prob, 1.0)),
                                 0.0))
        ent_ref[...] = ent.reshape(1, 1)


def _vq(x2, e2):
    out_shapes = (
        jax.ShapeDtypeStruct((_NS, _D), jnp.float32),
        jax.ShapeDtypeStruct((_NS, 1), jnp.float32),
        jax.ShapeDtypeStruct((1, _K), jnp.float32),
        jax.ShapeDtypeStruct((1, 1), jnp.float32),
    )
    e1 = e2.astype(jnp.bfloat16)
    e2r = e2 - e1.astype(jnp.float32)
    eb2 = e2r.astype(jnp.bfloat16)
    eb3 = (e2r - eb2.astype(jnp.float32)).astype(jnp.bfloat16)
    espec = pl.BlockSpec((_K, _D), lambda i: (0, 0))
    return pl.pallas_call(
        _vq_kernel,
        grid=(_G,),
        in_specs=[
            pl.BlockSpec((_R, _D), lambda i: (i, 0)),
            espec, espec, espec,
        ],
        out_specs=(
            pl.BlockSpec((_R, _D), lambda i: (i, 0)),
            pl.BlockSpec((_R, 1), lambda i: (i, 0)),
            pl.BlockSpec((1, _K), lambda i: (0, 0)),
            pl.BlockSpec((1, 1), lambda i: (0, 0)),
        ),
        out_shape=out_shapes,
        compiler_params=pltpu.CompilerParams(
            dimension_semantics=("arbitrary",)),
    )(x2, e1, eb2, eb3)


def kernel(x0, embedding0):
    x2 = x0.reshape(_NS, _D)
    e2 = embedding0.reshape(_K, _D)
    o0, o1, _hist, ent = _vq(x2, e2)
    out0 = o0.reshape(_N, _S, _C, _D)
    out1 = o1.reshape(_N, _S, _C)
    entropy = ent[0, 0]
    return (out0, out1, out1, entropy)


# speedup vs baseline: 1.5432x; 1.1416x over previous
import jax
import jax.numpy as jnp
from jax.experimental import pallas as pl
from jax.experimental.pallas import tpu as pltpu

_N, _S, _C, _K, _D = 8, 576, 1, 1024, 256
_NS = _N * _S
_R = 768
_G = _NS // _R


def _vq_kernel(x_ref, e_ref, out0_ref, out1_ref, hist_ref, ent_ref):
    i = pl.program_id(0)
    x = x_ref[...]
    e = e_ref[...]

    e1 = e.astype(jnp.bfloat16)
    e2r = e - e1.astype(jnp.float32)
    e2 = e2r.astype(jnp.bfloat16)
    e3 = (e2r - e2.astype(jnp.float32)).astype(jnp.bfloat16)

    xe = jax.lax.dot_general(
        x.astype(jnp.bfloat16), e1,
        (((1,), (1,)), ((), ())), preferred_element_type=jnp.float32)
    s = -2.0 * xe

    iota = jax.lax.broadcasted_iota(jnp.int32, (_R, _K), 1)
    big = jnp.float32(3.0e38)

    cand = []
    for _ in range(3):
        m = jnp.min(s, axis=1)[:, None]
        ij = jnp.min(jnp.where(s == m, iota, _K), axis=1)[:, None]
        cand.append(ij)
        s = jnp.where(iota == ij, big, s)

    best_d = best_i = best_e = None
    for ij in cand:
        oh = (iota == ij).astype(jnp.bfloat16)
        dn = (((1,), (0,)), ((), ()))
        ev = (jax.lax.dot_general(oh, e1, dn, preferred_element_type=jnp.float32)
              + jax.lax.dot_general(oh, e2, dn, preferred_element_type=jnp.float32)
              ) + jax.lax.dot_general(oh, e3, dn, preferred_element_type=jnp.float32)
        diff = x - ev
        sq = diff * diff
        dp = sq[:, :128] + sq[:, 128:]
        acc = dp
        for j in range(1, 16):
            acc = acc + jnp.roll(dp, -8 * j, axis=1)
        b = acc + jnp.roll(acc, -4, axis=1)
        c = b + jnp.roll(b, -2, axis=1)
        t = c + jnp.roll(c, -1, axis=1)
        d2 = t[:, 0:1]
        d = d2 * jax.lax.rsqrt(d2)
        if best_d is None:
            best_d, best_i, best_e = d, ij, ev
        else:
            take = (d < best_d) | ((d == best_d) & (ij < best_i))
            best_d = jnp.where(take, d, best_d)
            best_i = jnp.where(take, ij, best_i)
            best_e = jnp.where(take, ev, best_e)

    out0_ref[...] = (best_e - x) + x
    out1_ref[...] = best_d * best_d

    ohw = (iota == best_i).astype(jnp.float32)
    h = jnp.sum(ohw, axis=0, keepdims=True)

    @pl.when(i == 0)
    def _():
        hist_ref[...] = h

    @pl.when(i != 0)
    def _():
        hist_ref[...] = hist_ref[...] + h

    @pl.when(i == _G - 1)
    def _():
        hist = hist_ref[...]
        prob = hist / jnp.float32(_NS)
        ent = -jnp.sum(jnp.where(hist > 0,
                                 prob * jnp.log(jnp.where(hist > 0, prob, 1.0)),
                                 0.0))
        ent_ref[...] = ent.reshape(1, 1)


def _vq(x2, e2):
    out_shapes = (
        jax.ShapeDtypeStruct((_NS, _D), jnp.float32),
        jax.ShapeDtypeStruct((_NS, 1), jnp.float32),
        jax.ShapeDtypeStruct((1, _K), jnp.float32),
        jax.ShapeDtypeStruct((1, 1), jnp.float32),
    )
    return pl.pallas_call(
        _vq_kernel,
        grid=(_G,),
        in_specs=[
            pl.BlockSpec((_R, _D), lambda i: (i, 0)),
            pl.BlockSpec((_K, _D), lambda i: (0, 0)),
        ],
        out_specs=(
            pl.BlockSpec((_R, _D), lambda i: (i, 0)),
            pl.BlockSpec((_R, 1), lambda i: (i, 0)),
            pl.BlockSpec((1, _K), lambda i: (0, 0)),
            pl.BlockSpec((1, 1), lambda i: (0, 0)),
        ),
        out_shape=out_shapes,
        compiler_params=pltpu.CompilerParams(
            dimension_semantics=("arbitrary",)),
    )(x2, e2)


def kernel(x0, embedding0):
    x2 = x0.reshape(_NS, _D)
    e2 = embedding0.reshape(_K, _D)
    o0, o1, _hist, ent = _vq(x2, e2)
    out0 = o0.reshape(_N, _S, _C, _D)
    out1 = o1.reshape(_N, _S, _C)
    entropy = ent[0, 0]
    return (out0, out1, out1, entropy)
